# baseline (speedup 1.0000x reference)
import functools

import jax
import jax.numpy as jnp
from jax import lax
from jax.experimental import pallas as pl
from jax.experimental.pallas import tpu as pltpu
from jax.experimental.pallas import tpu_sc as plsc

NUM_CORES = 2
NUM_SUBCORES = 16
LANES = 16
NUM_WORKERS = NUM_CORES * NUM_SUBCORES

BATCH = 16384
DIM = 64
B_PER_W = BATCH // NUM_WORKERS
CHUNK = 128
NCHUNK = B_PER_W // CHUNK
GROUPS = CHUNK // LANES


def _make_kernel():
    mesh = plsc.VectorSubcoreMesh(core_axis_name="c", subcore_axis_name="s")

    @functools.partial(
        pl.kernel,
        mesh=mesh,
        compiler_params=pltpu.CompilerParams(
            needs_layout_passes=False, use_tc_tiling_on_sc=False),
        out_type=jax.ShapeDtypeStruct((BATCH,), jnp.float32),
        scratch_types=[
            pltpu.VMEM((NCHUNK, CHUNK), jnp.int32),
            pltpu.VMEM((NCHUNK, CHUNK), jnp.int32),
            pltpu.VMEM((2, CHUNK, DIM), jnp.float32),
            pltpu.VMEM((2, CHUNK, DIM), jnp.float32),
            pltpu.VMEM((B_PER_W,), jnp.float32),
            pltpu.VMEM((LANES * LANES,), jnp.float32),
            pltpu.SemaphoreType.DMA,
            pltpu.SemaphoreType.DMA,
        ],
    )
    def bprmf_kernel(user_hbm, item_hbm, ut_hbm, it_hbm, out_hbm,
                     uidx, iidx, ubuf, ibuf, outv, part, usem, isem):
        cid = lax.axis_index("c")
        sid = lax.axis_index("s")
        wid = sid * NUM_CORES + cid
        base = wid * B_PER_W

        pltpu.sync_copy(user_hbm.at[wid], uidx)
        pltpu.sync_copy(item_hbm.at[wid], iidx)

        pltpu.async_copy(ut_hbm.at[uidx.at[0]], ubuf.at[0], usem)
        pltpu.async_copy(it_hbm.at[iidx.at[0]], ibuf.at[0], isem)

        for c in range(NCHUNK):
            slot = c % 2
            pltpu.make_async_copy(ut_hbm.at[uidx.at[c]], ubuf.at[slot],
                                  usem).wait()
            pltpu.make_async_copy(it_hbm.at[iidx.at[c]], ibuf.at[slot],
                                  isem).wait()
            if c + 1 < NCHUNK:
                pltpu.async_copy(ut_hbm.at[uidx.at[c + 1]],
                                 ubuf.at[1 - slot], usem)
                pltpu.async_copy(it_hbm.at[iidx.at[c + 1]],
                                 ibuf.at[1 - slot], isem)

            urows = ubuf.at[slot]
            irows = ibuf.at[slot]

            def group_body(g, carry, urows=urows, irows=irows, c=c):
                rows = g * LANES + lax.iota(jnp.int32, LANES)
                acc = jnp.zeros((LANES,), jnp.float32)
                for j in range(DIM):
                    col = jnp.full((LANES,), j, jnp.int32)
                    u = plsc.load_gather(urows, [rows, col])
                    v = plsc.load_gather(irows, [rows, col])
                    acc = acc + u * v
                outv[pl.ds(c * CHUNK + g * LANES, LANES)] = acc
                return carry

            lax.fori_loop(0, GROUPS, group_body, 0)

        pltpu.sync_copy(outv, out_hbm.at[pl.ds(base, B_PER_W)])

    return bprmf_kernel


_BPRMF = _make_kernel()


@jax.jit
def kernel(user, item, user_table, item_table):
    user3 = user.reshape(NUM_WORKERS, NCHUNK, CHUNK)
    item3 = item.reshape(NUM_WORKERS, NCHUNK, CHUNK)
    return _BPRMF(user3, item3, user_table, item_table)
